# baseline (speedup 1.0000x reference)
import functools

import jax
import jax.numpy as jnp
import numpy as np
from jax import lax
from jax.experimental import pallas as pl
from jax.experimental.pallas import tpu as pltpu
from jax.experimental.pallas import tpu_sc as plsc

_N_NODES = 10000
_D = 128
_G = 64
_NC = 2
_NS = 16
_NW = _NC * _NS
_CHUNK = 320
_N_PAD = _CHUNK * _NW
_IDXW = 16
_NJ = _CHUNK // _IDXW
_HW = _G * 16 + 16
_GA = _G + 8
_BS = 80
_NB = _CHUNK // _BS
_ZX = np.zeros((_GA, _D), np.float32)
_ZC = np.zeros((_G + 1, 16), np.float32)


@functools.partial(
    pl.kernel,
    out_type=[
        jax.ShapeDtypeStruct((_NC, _GA, _D), jnp.float32),
        jax.ShapeDtypeStruct((_NW, _G + 1, 16), jnp.float32),
    ],
    mesh=plsc.VectorSubcoreMesh(core_axis_name="c", subcore_axis_name="s"),
    compiler_params=pltpu.CompilerParams(needs_layout_passes=False),
    scratch_types=[
        pltpu.VMEM((_CHUNK, _D), jnp.float32),
        pltpu.VMEM((_CHUNK,), jnp.int32),
        pltpu.VMEM((_G + 1, 16), jnp.float32),
        pltpu.VMEM_SHARED((_GA, _D), jnp.float32),
        pltpu.SemaphoreType.DMA,
        pltpu.SemaphoreType.DMA,
    ],
)
def _sc_segment(x_hbm, b_hbm, zx_hbm, zc_hbm, out_x, out_c,
                xbuf, idx, hist, acc_x, ssem, csem):
    cid = lax.axis_index("c")
    sid = lax.axis_index("s")
    wid = sid * _NC + cid
    base = wid * _CHUNK
    valid = _N_NODES - base

    @pl.when(sid == 0)
    def _zero():
        pltpu.sync_copy(zx_hbm, acc_x)

    @pl.when(wid < _NW - 1)
    def _stage_ids():
        pltpu.sync_copy(b_hbm.at[pl.ds(base, _CHUNK)], idx)

    @pl.when(wid == _NW - 1)
    def _stage_ids_tail():
        pltpu.sync_copy(b_hbm.at[pl.ds(base, _N_NODES - (_NW - 1) * _CHUNK)],
                        idx.at[pl.ds(0, _N_NODES - (_NW - 1) * _CHUNK)])

    pltpu.sync_copy(zc_hbm, hist)

    def stage(b):
        offs = pl.multiple_of(jnp.minimum(base + b * _BS, _N_NODES - _BS), 8)
        return pltpu.async_copy(x_hbm.at[pl.ds(offs, _BS)],
                                xbuf.at[pl.ds(b * _BS, _BS)], ssem)

    stage_descs = [stage(0)]

    lanes = lax.iota(jnp.int32, 16)
    one16 = jnp.ones((16,), jnp.float32)

    plsc.subcore_barrier()

    descs = []
    for b in range(_NB):
        stage_descs[b].wait()
        if b + 1 < _NB:
            stage_descs.append(stage(b + 1))
        for k in range(_BS // _IDXW):
            j = b * (_BS // _IDXW) + k
            bv = idx[pl.ds(j * _IDXW, _IDXW)]
            ok = (j * _IDXW + lanes) < valid
            bvx = jnp.where(ok, bv, _G)
            descs.append(pltpu.async_copy(xbuf.at[pl.ds(j * _IDXW, _IDXW)],
                                          acc_x.at[bvx], csem, add=True))
            plsc.addupdate_scatter(hist, [bvx, lanes], one16)

    pltpu.sync_copy(hist, out_c.at[wid])
    for d in descs:
        d.wait()

    plsc.subcore_barrier()

    @pl.when(sid == 0)
    def _dump():
        pltpu.sync_copy(acc_x, out_x.at[cid])


def _tc_body(px_ref, pc_ref, u_ref, w0_ref, b0_ref, w1_ref, b1_ref, o_ref):
    sums = px_ref[0, 0:_G] + px_ref[1, 0:_G]
    cnt = jnp.sum(jnp.sum(pc_ref[:, 0:_G, :], axis=0), axis=1, keepdims=True)
    mean = sums / jnp.maximum(cnt, 1.0)
    w0 = w0_ref[...]
    h = (jnp.dot(u_ref[...], w0[0:_D, :], preferred_element_type=jnp.float32,
                 precision=lax.Precision.HIGHEST)
         + jnp.dot(mean, w0[_D:2 * _D, :], preferred_element_type=jnp.float32,
                   precision=lax.Precision.HIGHEST)
         + b0_ref[...])
    h = jnp.maximum(h, 0.0)
    o_ref[...] = (jnp.dot(h, w1_ref[...], preferred_element_type=jnp.float32,
                          precision=lax.Precision.HIGHEST) + b1_ref[...])


_tc_mlp = pl.pallas_call(
    _tc_body,
    out_shape=jax.ShapeDtypeStruct((_G, _D), jnp.float32),
)


def kernel(x, edge_index, edge_attr, u, batch, W0, b0, W1, b1):
    del edge_index, edge_attr
    part_x, part_c = _sc_segment(x, batch, _ZX, _ZC)
    return _tc_mlp(part_x, part_c, u, W0, b0.reshape(1, _D), W1,
                   b1.reshape(1, _D))


# speedup vs baseline: 1.0328x; 1.0328x over previous
import functools

import jax
import jax.numpy as jnp
import numpy as np
from jax import lax
from jax.experimental import pallas as pl
from jax.experimental.pallas import tpu as pltpu
from jax.experimental.pallas import tpu_sc as plsc

_N_NODES = 10000
_D = 128
_G = 64
_NC = 2
_NS = 16
_NW = _NC * _NS
_CHUNK = 320
_N_PAD = _CHUNK * _NW
_IDXW = 16
_NJ = _CHUNK // _IDXW
_HW = _G * 16 + 16
_GA = _G + 8
_SW = 64
_NJ2 = _CHUNK // _SW
_ZX = np.zeros((_GA, _D), np.float32)
_ZC = np.zeros((_G + 1, 16), np.float32)


@functools.partial(
    pl.kernel,
    out_type=[
        jax.ShapeDtypeStruct((_NC, _GA, _D), jnp.float32),
        jax.ShapeDtypeStruct((_NW, _G + 1, 16), jnp.float32),
    ],
    mesh=plsc.VectorSubcoreMesh(core_axis_name="c", subcore_axis_name="s"),
    compiler_params=pltpu.CompilerParams(needs_layout_passes=False),
    scratch_types=[
        pltpu.VMEM((_CHUNK, _D), jnp.float32),
        pltpu.VMEM((_NJ2, 1, _SW), jnp.int32),
        pltpu.VMEM((_G + 1, 16), jnp.float32),
        pltpu.VMEM_SHARED((_GA, _D), jnp.float32),
        pltpu.SemaphoreType.DMA,
    ],
)
def _sc_segment(x_hbm, b_hbm, zx_hbm, zc_hbm, out_x, out_c,
                xbuf, idx, hist, acc_x, csem):
    cid = lax.axis_index("c")
    sid = lax.axis_index("s")
    wid = sid * _NC + cid
    base = wid * _CHUNK

    @pl.when(sid == 0)
    def _zero():
        pltpu.sync_copy(zx_hbm, acc_x)

    pltpu.sync_copy(b_hbm.at[wid], idx)

    @pl.when(wid < _NW - 1)
    def _stage_main():
        pltpu.sync_copy(x_hbm.at[pl.ds(base, _CHUNK)], xbuf)

    @pl.when(wid == _NW - 1)
    def _stage_tail():
        pltpu.sync_copy(x_hbm.at[pl.ds(base, _N_NODES - (_NW - 1) * _CHUNK)],
                        xbuf.at[pl.ds(0, _N_NODES - (_NW - 1) * _CHUNK)])

    pltpu.sync_copy(zc_hbm, hist)

    lanes = lax.iota(jnp.int32, 16)
    one16 = jnp.ones((16,), jnp.float32)

    plsc.subcore_barrier()

    descs = []
    for j in range(_NJ2):
        descs.append(pltpu.async_copy(xbuf.at[pl.ds(j * _SW, _SW)],
                                      acc_x.at[idx.at[j, 0]], csem, add=True))
        for k in range(_SW // 16):
            bv = idx[j, 0, pl.ds(k * 16, 16)]
            plsc.addupdate_scatter(hist, [bv, lanes], one16)

    pltpu.sync_copy(hist, out_c.at[wid])
    for d in descs:
        d.wait()

    plsc.subcore_barrier()

    @pl.when(sid == 0)
    def _dump():
        pltpu.sync_copy(acc_x, out_x.at[cid])


def _tc_body(px_ref, pc_ref, u_ref, w0_ref, b0_ref, w1_ref, b1_ref, o_ref):
    sums = px_ref[0, 0:_G] + px_ref[1, 0:_G]
    cnt = jnp.sum(jnp.sum(pc_ref[:, 0:_G, :], axis=0), axis=1, keepdims=True)
    mean = sums / jnp.maximum(cnt, 1.0)
    w0 = w0_ref[...]
    h = (jnp.dot(u_ref[...], w0[0:_D, :], preferred_element_type=jnp.float32,
                 precision=lax.Precision.HIGHEST)
         + jnp.dot(mean, w0[_D:2 * _D, :], preferred_element_type=jnp.float32,
                   precision=lax.Precision.HIGHEST)
         + b0_ref[...])
    h = jnp.maximum(h, 0.0)
    o_ref[...] = (jnp.dot(h, w1_ref[...], preferred_element_type=jnp.float32,
                          precision=lax.Precision.HIGHEST) + b1_ref[...])


_tc_mlp = pl.pallas_call(
    _tc_body,
    out_shape=jax.ShapeDtypeStruct((_G, _D), jnp.float32),
)


def kernel(x, edge_index, edge_attr, u, batch, W0, b0, W1, b1):
    del edge_index, edge_attr
    b3 = jnp.pad(batch, (0, _N_PAD - _N_NODES),
                 constant_values=_G).reshape(_NW, _NJ2, 1, _SW)
    part_x, part_c = _sc_segment(x, b3, _ZX, _ZC)
    return _tc_mlp(part_x, part_c, u, W0, b0.reshape(1, _D), W1,
                   b1.reshape(1, _D))
